# baseline (speedup 1.0000x reference)
import functools

import jax
import jax.numpy as jnp
from jax import lax
from jax.experimental import pallas as pl
from jax.experimental.pallas import tpu as pltpu
from jax.experimental.pallas import tpu_sc as plsc

L = 16


def _make_lookup(nw, nc, b, f, d):
    bpw = b // nw
    fpad = ((f + L - 1) // L) * L
    mesh = plsc.VectorSubcoreMesh(core_axis_name="c", subcore_axis_name="s")

    @functools.partial(
        pl.kernel,
        out_type=jax.ShapeDtypeStruct((b, f, d), jnp.float32),
        mesh=mesh,
        scratch_types=[
            pltpu.VMEM((bpw, fpad), jnp.int32),
            pltpu.VMEM((4, f, d), jnp.float32),
            pltpu.SemaphoreType.DMA,
            pltpu.SemaphoreType.DMA,
            pltpu.SemaphoreType.DMA,
            pltpu.SemaphoreType.DMA,
            pltpu.SemaphoreType.DMA,
        ],
    )
    def lookup_kernel(
        idx_hbm, table_hbm, out_hbm, idx_v, rows_v, si, s0, s1, s2, s3
    ):
        wid = lax.axis_index("s") * nc + lax.axis_index("c")
        b0 = wid * bpw

        def stage(bl, c):
            pltpu.make_async_copy(
                idx_hbm.at[b0 + bl], idx_v.at[bl, pl.ds(0, f)], si
            ).start()
            return c

        lax.fori_loop(0, bpw, stage, 0)

        def stage_wait(bl, c):
            pltpu.make_async_copy(
                idx_hbm.at[b0 + bl], idx_v.at[bl, pl.ds(0, f)], si
            ).wait()
            return c

        lax.fori_loop(0, bpw, stage_wait, 0)

        def fire(bl, slot, sem):
            for r0 in range(0, fpad, L):
                vec = idx_v[bl, pl.ds(r0, L)]
                for l in range(L):
                    if r0 + l < f:
                        pltpu.make_async_copy(
                            table_hbm.at[vec[l]], rows_v.at[slot, r0 + l], sem
                        ).start()

        def drain_write(bl, slot, sem):
            for r in range(f):
                pltpu.make_async_copy(
                    table_hbm.at[0], rows_v.at[slot, r], sem
                ).wait()
            pltpu.sync_copy(rows_v.at[slot], out_hbm.at[b0 + bl])

        sems = [s0, s1, s2, s3]
        fire(0, 0, s0)
        fire(1, 1, s1)
        fire(2, 2, s2)

        def body(blq, c):
            bl = 4 * blq
            fire(bl + 3, 3, s3)
            for q in range(4):
                drain_write(bl + q, q, sems[q])
                if q < 3:

                    @pl.when(bl + q + 4 < bpw)
                    def _(q=q):
                        fire(bl + q + 4, q, sems[q])

            return c

        lax.fori_loop(0, bpw // 4, body, 0)

    return lookup_kernel


def kernel(indices, weight):
    b, f = indices.shape
    v, d = weight.shape
    info = plsc.get_sparse_core_info()
    nc, ns = info.num_cores, info.num_subcores
    nw = nc * ns
    assert b % (2 * nw) == 0
    return _make_lookup(nw, nc, b, f, d)(indices, weight)


# speedup vs baseline: 1.1070x; 1.1070x over previous
import functools

import jax
import jax.numpy as jnp
from jax import lax
from jax.experimental import pallas as pl
from jax.experimental.pallas import tpu as pltpu
from jax.experimental.pallas import tpu_sc as plsc

L = 16


def _make_lookup(nw, nc, b, f, d):
    bpw = b // nw
    fpad = ((f + L - 1) // L) * L
    mesh = plsc.VectorSubcoreMesh(core_axis_name="c", subcore_axis_name="s")

    @functools.partial(
        pl.kernel,
        out_type=jax.ShapeDtypeStruct((b, f, d), jnp.float32),
        mesh=mesh,
        scratch_types=[
            pltpu.VMEM((bpw, fpad), jnp.int32),
            pltpu.VMEM((2, f, d), jnp.float32),
            pltpu.SemaphoreType.DMA,
            pltpu.SemaphoreType.DMA,
            pltpu.SemaphoreType.DMA,
        ],
    )
    def lookup_kernel(idx_hbm, table_hbm, out_hbm, idx_v, rows_v, si, s0, s1):
        wid = lax.axis_index("s") * nc + lax.axis_index("c")
        b0 = wid * bpw

        def stage(bl, c):
            pltpu.make_async_copy(
                idx_hbm.at[b0 + bl], idx_v.at[bl, pl.ds(0, f)], si
            ).start()
            return c

        lax.fori_loop(0, bpw, stage, 0)

        def stage_wait(bl, c):
            pltpu.make_async_copy(
                idx_hbm.at[b0 + bl], idx_v.at[bl, pl.ds(0, f)], si
            ).wait()
            return c

        lax.fori_loop(0, bpw, stage_wait, 0)

        def fire(bl, slot, sem):
            for r0 in range(0, fpad, L):
                vec = idx_v[bl, pl.ds(r0, L)]
                for l in range(L):
                    if r0 + l < f:
                        pltpu.make_async_copy(
                            table_hbm.at[vec[l]], rows_v.at[slot, r0 + l], sem
                        ).start()

        def drain_write(bl, slot, sem):
            for r in range(f):
                pltpu.make_async_copy(
                    table_hbm.at[0], rows_v.at[slot, r], sem
                ).wait()
            pltpu.sync_copy(rows_v.at[slot], out_hbm.at[b0 + bl])

        fire(0, 0, s0)

        def body(blp, c):
            bl0 = 2 * blp
            fire(bl0 + 1, 1, s1)
            drain_write(bl0, 0, s0)

            @pl.when(bl0 + 2 < bpw)
            def _():
                fire(bl0 + 2, 0, s0)

            drain_write(bl0 + 1, 1, s1)
            return c

        lax.fori_loop(0, bpw // 2, body, 0)

    return lookup_kernel


def kernel(indices, weight):
    b, f = indices.shape
    v, d = weight.shape
    info = plsc.get_sparse_core_info()
    nc, ns = info.num_cores, info.num_subcores
    nw = nc * ns
    assert b % (2 * nw) == 0
    return _make_lookup(nw, nc, b, f, d)(indices, weight)
